# baseline (speedup 1.0000x reference)
import functools
import math

import jax
import jax.numpy as jnp
from jax import lax
from jax.experimental import pallas as pl
from jax.experimental.pallas import tpu as pltpu

_FACTOR = 0.0005
_QB = 512
_KB = 512
_KSP = 128


def _body(n_top, U, B, L, H, D, scale,
          q_hbm, k_hbm, v_hbm, ks_ref, out_hbm,
          qv, kv, vv, ov, m_ref, att_ref, selq_ref, idx_ref,
          insem, outsem):
    L_K = L
    NB = L // _QB
    N = B * H
    f32 = jnp.float32
    dot = functools.partial(
        lax.dot_general,
        preferred_element_type=f32,
        precision=lax.Precision.HIGHEST,
    )

    i = pl.program_id(0)
    buf = lax.rem(i, 2)
    nbuf = 1 - buf
    b = i // H
    h = lax.rem(i, H)

    def in_copies(step, slot):
        bb = step // H
        hh = lax.rem(step, H)
        return (
            pltpu.make_async_copy(q_hbm.at[bb, :, hh, :], qv.at[slot],
                                  insem.at[slot, 0]),
            pltpu.make_async_copy(k_hbm.at[bb, :, hh, :], kv.at[slot],
                                  insem.at[slot, 1]),
            pltpu.make_async_copy(v_hbm.at[bb, :, hh, :], vv.at[slot],
                                  insem.at[slot, 2]),
        )

    @pl.when(i == 0)
    def _():
        for c in in_copies(i, buf):
            c.start()

    @pl.when(i + 1 < N)
    def _():
        for c in in_copies(i + 1, nbuf):
            c.start()

    cq, ck, cv = in_copies(i, buf)

    cq.wait()
    ks = ks_ref[0, 0]
    for ib in range(NB):
        qb = qv[buf, pl.ds(ib * _QB, _QB), :]
        sc = lax.dot_general(qb, ks, (((1,), (1,)), ((), ())),
                             preferred_element_type=f32)
        lane = lax.broadcasted_iota(jnp.int32, (_QB, _KSP), 1)
        valid = lane < U
        mx = jnp.max(jnp.where(valid, sc, -jnp.inf), axis=1)
        sm = jnp.sum(jnp.where(valid, sc, 0.0), axis=1)
        m_ref[ib, :] = mx - sm / L_K

    fid = (lax.broadcasted_iota(jnp.int32, (NB, _QB), 0) * _QB
           + lax.broadcasted_iota(jnp.int32, (NB, _QB), 1))

    def tk_body(t, carry):
        mc = m_ref[...]
        m = jnp.max(mc)
        idx = jnp.min(jnp.where(mc == m, fid, jnp.int32(2 ** 30)))
        idx_ref[t] = idx
        m_ref[...] = jnp.where(fid == idx, -jnp.inf, mc)
        return carry

    lax.fori_loop(0, n_top, tk_body, 0)

    SELP = selq_ref.shape[0]
    selq_ref[...] = jnp.zeros((SELP, D), f32)
    for t in range(n_top):
        it = idx_ref[t]
        selq_ref[pl.ds(t, 1), :] = qv[buf, pl.ds(it, 1), :]
    sq = selq_ref[...]
    ck.wait()
    for ib in range(L_K // _KB):
        kb = kv[buf, pl.ds(ib * _KB, _KB), :]
        att_ref[:, pl.ds(ib * _KB, _KB)] = (
            dot(sq, kb, (((1,), (1,)), ((), ()))) * scale)
    for r in range(SELP // 8):
        a = att_ref[pl.ds(r * 8, 8), :]
        a = a - jnp.max(a, axis=1, keepdims=True)
        e = jnp.exp(a)
        att_ref[pl.ds(r * 8, 8), :] = e / jnp.sum(e, axis=1, keepdims=True)
    cv.wait()
    ctx = jnp.zeros((SELP, D), f32)
    for ib in range(L_K // _KB):
        ab = att_ref[:, pl.ds(ib * _KB, _KB)]
        vb = vv[buf, pl.ds(ib * _KB, _KB), :]
        ctx = ctx + dot(ab, vb, (((1,), (0,)), ((), ())))

    def out_copy(slot, bb, hh):
        return pltpu.make_async_copy(ov.at[slot], out_hbm.at[bb, :, hh, :],
                                     outsem.at[slot])

    @pl.when(i >= 2)
    def _():
        out_copy(buf, b, h).wait()

    for ib in range(NB):
        ov[buf, pl.ds(ib * _QB, _QB), :] = jnp.zeros((_QB, D), f32)
    for t in range(n_top):
        it = idx_ref[t]
        ov[buf, pl.ds(it, 1), :] = ctx[t:t + 1, :]
    out_copy(buf, b, h).start()

    @pl.when(i == N - 1)
    def _():
        out_copy(buf, b, h).wait()
        @pl.when(N >= 2)
        def _():
            out_copy(nbuf, b, h).wait()


def kernel(queries, keys, values):
    B, L, H, D = queries.shape
    L_K = keys.shape[1]
    scale = 1.0 / math.sqrt(D)
    n_top = max(int(L * _FACTOR * math.log(L_K)), 1)
    U = min(n_top, L_K)
    SELP = (n_top + 7) // 8 * 8

    perm = jax.random.permutation(jax.random.key(42), L_K)[:U]
    ksamp = jnp.transpose(keys[:, perm], (0, 2, 1, 3))
    ksamp = jnp.pad(ksamp, ((0, 0), (0, 0), (0, _KSP - U), (0, 0)))

    any_spec = pl.BlockSpec(memory_space=pl.ANY)
    return pl.pallas_call(
        functools.partial(_body, n_top, U, B, L, H, D, scale),
        grid=(B * H,),
        in_specs=[
            any_spec,
            any_spec,
            any_spec,
            pl.BlockSpec((1, 1, _KSP, D), lambda i: (i // H, i % H, 0, 0)),
        ],
        out_specs=any_spec,
        out_shape=jax.ShapeDtypeStruct((B, L, H, D), jnp.float32),
        scratch_shapes=[
            pltpu.VMEM((2, L, D), jnp.float32),
            pltpu.VMEM((2, L, D), jnp.float32),
            pltpu.VMEM((2, L, D), jnp.float32),
            pltpu.VMEM((2, L, D), jnp.float32),
            pltpu.VMEM((L // _QB, _QB), jnp.float32),
            pltpu.VMEM((SELP, L_K), jnp.float32),
            pltpu.VMEM((SELP, D), jnp.float32),
            pltpu.SMEM((n_top,), jnp.int32),
            pltpu.SemaphoreType.DMA((2, 3)),
            pltpu.SemaphoreType.DMA((2,)),
        ],
        compiler_params=pltpu.CompilerParams(
            dimension_semantics=("arbitrary",),
        ),
    )(queries, keys, values, ksamp)


# speedup vs baseline: 1.5225x; 1.5225x over previous
import functools
import math

import jax
import jax.numpy as jnp
from jax import lax
from jax.experimental import pallas as pl
from jax.experimental.pallas import tpu as pltpu

_FACTOR = 0.0005
_CB = 512
_KSP = 128


def _body(n_top, U, B, L, H, D, scale,
          q_hbm, k_hbm, v_hbm, ks_ref, out_hbm,
          qv, kv, vv, ov, m_ref, att_ref, oh_ref, idx_ref,
          insem, outsem):
    L_K = L
    NC = L // _CB
    N = B * H
    f32 = jnp.float32
    SELP = att_ref.shape[0]
    hi = functools.partial(
        lax.dot_general,
        preferred_element_type=f32,
        precision=lax.Precision.HIGHEST,
    )

    i = pl.program_id(0)
    buf = lax.rem(i, 2)
    nbuf = 1 - buf
    b = i // H
    h = lax.rem(i, H)

    def in_copies(step, slot):
        bb = step // H
        hh = lax.rem(step, H)
        return (
            pltpu.make_async_copy(q_hbm.at[bb, hh], qv.at[slot],
                                  insem.at[slot, 0]),
            pltpu.make_async_copy(k_hbm.at[bb, hh], kv.at[slot],
                                  insem.at[slot, 1]),
            pltpu.make_async_copy(v_hbm.at[bb, hh], vv.at[slot],
                                  insem.at[slot, 2]),
        )

    @pl.when(i == 0)
    def _():
        for c in in_copies(i, buf):
            c.start()

    @pl.when(i + 1 < N)
    def _():
        for c in in_copies(i + 1, nbuf):
            c.start()

    cq, ck, cv = in_copies(i, buf)

    cq.wait()
    ks = ks_ref[0, 0]
    row = lax.broadcasted_iota(jnp.int32, (_KSP, _CB), 0)
    valid = row < U
    for c in range(NC):
        qc = qv[buf, :, pl.ds(c * _CB, _CB)]
        sc = lax.dot_general(ks, qc, (((1,), (0,)), ((), ())),
                             preferred_element_type=f32)
        mx = jnp.max(jnp.where(valid, sc, -jnp.inf), axis=0)
        sm = jnp.sum(jnp.where(valid, sc, 0.0), axis=0)
        m_ref[c, :] = mx - sm / L_K

    fid = (lax.broadcasted_iota(jnp.int32, (NC, _CB), 0) * _CB
           + lax.broadcasted_iota(jnp.int32, (NC, _CB), 1))

    def tk_body(t, carry):
        mc = m_ref[...]
        m = jnp.max(mc)
        idx = jnp.min(jnp.where(mc == m, fid, jnp.int32(2 ** 30)))
        idx_ref[t] = idx
        m_ref[...] = jnp.where(fid == idx, -jnp.inf, mc)
        return carry

    lax.fori_loop(0, n_top, tk_body, 0)

    for c in range(NC):
        oh_ref[pl.ds(c * _CB, _CB), :] = jnp.zeros((_CB, SELP), f32)
    for t in range(n_top):
        it = idx_ref[t]
        oh_ref[pl.ds(it, 1), t:t + 1] = jnp.ones((1, 1), f32)
    oh = oh_ref[...]
    selq = hi(qv[buf], oh, (((1,), (0,)), ((), ())))

    ck.wait()
    for c in range(NC):
        kc = kv[buf, :, pl.ds(c * _CB, _CB)]
        att_ref[:, pl.ds(c * _CB, _CB)] = (
            hi(selq, kc, (((0,), (0,)), ((), ()))) * scale)
    for r in range(SELP // 8):
        a = att_ref[pl.ds(r * 8, 8), :]
        a = a - jnp.max(a, axis=1, keepdims=True)
        e = jnp.exp(a)
        att_ref[pl.ds(r * 8, 8), :] = e / jnp.sum(e, axis=1, keepdims=True)
    cv.wait()
    ctx = hi(vv[buf], att_ref[...], (((1,), (1,)), ((), ())))

    def out_copy(slot, bb, hh):
        return pltpu.make_async_copy(ov.at[slot], out_hbm.at[bb, hh],
                                     outsem.at[slot])

    @pl.when(i >= 2)
    def _():
        out_copy(buf, b, h).wait()

    for c in range(NC):
        ohc = oh_ref[pl.ds(c * _CB, _CB), :]
        ov[buf, :, pl.ds(c * _CB, _CB)] = (
            hi(ctx, ohc, (((1,), (1,)), ((), ()))))
    out_copy(buf, b, h).start()

    @pl.when(i == N - 1)
    def _():
        out_copy(buf, b, h).wait()
        @pl.when(N >= 2)
        def _():
            out_copy(nbuf, b, h).wait()


def kernel(queries, keys, values):
    B, L, H, D = queries.shape
    L_K = keys.shape[1]
    scale = 1.0 / math.sqrt(D)
    n_top = max(int(L * _FACTOR * math.log(L_K)), 1)
    U = min(n_top, L_K)
    SELP = (n_top + 7) // 8 * 8

    qT = jnp.transpose(queries, (0, 2, 3, 1))
    kT = jnp.transpose(keys, (0, 2, 3, 1))
    vT = jnp.transpose(values, (0, 2, 3, 1))

    perm = jax.random.permutation(jax.random.key(42), L_K)[:U]
    ksamp = jnp.transpose(keys[:, perm], (0, 2, 1, 3))
    ksamp = jnp.pad(ksamp, ((0, 0), (0, 0), (0, _KSP - U), (0, 0)))

    any_spec = pl.BlockSpec(memory_space=pl.ANY)
    outT = pl.pallas_call(
        functools.partial(_body, n_top, U, B, L, H, D, scale),
        grid=(B * H,),
        in_specs=[
            any_spec,
            any_spec,
            any_spec,
            pl.BlockSpec((1, 1, _KSP, D), lambda i: (i // H, i % H, 0, 0)),
        ],
        out_specs=any_spec,
        out_shape=jax.ShapeDtypeStruct((B, H, D, L), jnp.float32),
        scratch_shapes=[
            pltpu.VMEM((2, D, L), jnp.float32),
            pltpu.VMEM((2, D, L), jnp.float32),
            pltpu.VMEM((2, D, L), jnp.float32),
            pltpu.VMEM((2, D, L), jnp.float32),
            pltpu.VMEM((L // _CB, _CB), jnp.float32),
            pltpu.VMEM((SELP, L_K), jnp.float32),
            pltpu.VMEM((L, SELP), jnp.float32),
            pltpu.SMEM((n_top,), jnp.int32),
            pltpu.SemaphoreType.DMA((2, 3)),
            pltpu.SemaphoreType.DMA((2,)),
        ],
        compiler_params=pltpu.CompilerParams(
            dimension_semantics=("arbitrary",),
        ),
    )(qT, kT, vT, ksamp)
    return jnp.transpose(outT, (0, 3, 1, 2))


# speedup vs baseline: 2.4490x; 1.6085x over previous
import functools
import math

import jax
import jax.numpy as jnp
from jax import lax
from jax.experimental import pallas as pl
from jax.experimental.pallas import tpu as pltpu

_FACTOR = 0.0005
_CB = 512
_KSP = 128


def _body(n_top, U, B, L, H, D, scale,
          q_hbm, k_hbm, v_hbm, perm_ref, out_hbm,
          qv, kv, vv, ov, m_ref, att_ref, oh_ref, ohp_ref, idx_ref,
          insem, outsem):
    L_K = L
    NC = L // _CB
    N = B * H
    f32 = jnp.float32
    SELP = att_ref.shape[0]
    dot = functools.partial(lax.dot_general, preferred_element_type=f32)

    i = pl.program_id(0)
    buf = lax.rem(i, 2)
    nbuf = 1 - buf
    b = i // H
    h = lax.rem(i, H)

    def in_copies(step, slot):
        bb = step // H
        hh = lax.rem(step, H)
        return (
            pltpu.make_async_copy(q_hbm.at[bb, hh], qv.at[slot],
                                  insem.at[slot, 0]),
            pltpu.make_async_copy(k_hbm.at[bb, hh], kv.at[slot],
                                  insem.at[slot, 1]),
            pltpu.make_async_copy(v_hbm.at[bb, hh], vv.at[slot],
                                  insem.at[slot, 2]),
        )

    @pl.when(i == 0)
    def _():
        for c in in_copies(i, buf):
            c.start()
        for c in range(NC):
            ohp_ref[pl.ds(c * _CB, _CB), :] = jnp.zeros((_CB, _KSP), f32)
        for s in range(U):
            ps = perm_ref[s]
            ohp_ref[pl.ds(ps, 1), s:s + 1] = jnp.ones((1, 1), f32)

    @pl.when(i + 1 < N)
    def _():
        for c in in_copies(i + 1, nbuf):
            c.start()

    cq, ck, cv = in_copies(i, buf)

    cq.wait()
    ck.wait()
    ksT = dot(kv[buf], ohp_ref[...], (((1,), (0,)), ((), ())))
    row = lax.broadcasted_iota(jnp.int32, (_KSP, _CB), 0)
    valid = row < U
    for c in range(NC):
        qc = qv[buf, :, pl.ds(c * _CB, _CB)]
        sc = dot(ksT, qc, (((0,), (0,)), ((), ())))
        mx = jnp.max(jnp.where(valid, sc, -jnp.inf), axis=0)
        sm = jnp.sum(jnp.where(valid, sc, 0.0), axis=0)
        m_ref[c, :] = mx - sm / L_K

    fid = (lax.broadcasted_iota(jnp.int32, (NC, _CB), 0) * _CB
           + lax.broadcasted_iota(jnp.int32, (NC, _CB), 1))

    def tk_body(t, carry):
        mc = m_ref[...]
        m = jnp.max(mc)
        idx = jnp.min(jnp.where(mc == m, fid, jnp.int32(2 ** 30)))
        idx_ref[t] = idx
        m_ref[...] = jnp.where(fid == idx, -jnp.inf, mc)
        return carry

    lax.fori_loop(0, n_top, tk_body, 0)

    for c in range(NC):
        oh_ref[pl.ds(c * _CB, _CB), :] = jnp.zeros((_CB, SELP), f32)
    for t in range(n_top):
        it = idx_ref[t]
        oh_ref[pl.ds(it, 1), t:t + 1] = jnp.ones((1, 1), f32)
    oh = oh_ref[...]
    selq = dot(qv[buf], oh, (((1,), (0,)), ((), ())))

    for c in range(NC):
        kc = kv[buf, :, pl.ds(c * _CB, _CB)]
        att_ref[:, pl.ds(c * _CB, _CB)] = (
            dot(selq, kc, (((0,), (0,)), ((), ()))) * scale)
    for r in range(SELP // 8):
        a = att_ref[pl.ds(r * 8, 8), :]
        a = a - jnp.max(a, axis=1, keepdims=True)
        e = jnp.exp(a)
        att_ref[pl.ds(r * 8, 8), :] = e / jnp.sum(e, axis=1, keepdims=True)
    cv.wait()
    ctx = dot(vv[buf], att_ref[...], (((1,), (1,)), ((), ())))

    def out_copy(slot, bb, hh):
        return pltpu.make_async_copy(ov.at[slot], out_hbm.at[bb, hh],
                                     outsem.at[slot])

    @pl.when(i >= 2)
    def _():
        out_copy(buf, b, h).wait()

    for c in range(NC):
        ohc = oh_ref[pl.ds(c * _CB, _CB), :]
        ov[buf, :, pl.ds(c * _CB, _CB)] = (
            dot(ctx, ohc, (((1,), (1,)), ((), ()))))
    out_copy(buf, b, h).start()

    @pl.when(i == N - 1)
    def _():
        out_copy(buf, b, h).wait()
        @pl.when(N >= 2)
        def _():
            out_copy(nbuf, b, h).wait()


def kernel(queries, keys, values):
    B, L, H, D = queries.shape
    L_K = keys.shape[1]
    scale = 1.0 / math.sqrt(D)
    n_top = max(int(L * _FACTOR * math.log(L_K)), 1)
    U = min(n_top, L_K)
    SELP = (n_top + 7) // 8 * 8

    qT = jnp.transpose(queries, (0, 2, 3, 1))
    kT = jnp.transpose(keys, (0, 2, 3, 1))
    vT = jnp.transpose(values, (0, 2, 3, 1))

    perm = jax.random.permutation(jax.random.key(42), L_K)[:U]
    perm = jnp.pad(perm.astype(jnp.int32), (0, _KSP - U))

    any_spec = pl.BlockSpec(memory_space=pl.ANY)
    outT = pl.pallas_call(
        functools.partial(_body, n_top, U, B, L, H, D, scale),
        grid=(B * H,),
        in_specs=[
            any_spec,
            any_spec,
            any_spec,
            pl.BlockSpec(memory_space=pltpu.SMEM),
        ],
        out_specs=any_spec,
        out_shape=jax.ShapeDtypeStruct((B, H, D, L), jnp.float32),
        scratch_shapes=[
            pltpu.VMEM((2, D, L), jnp.float32),
            pltpu.VMEM((2, D, L), jnp.float32),
            pltpu.VMEM((2, D, L), jnp.float32),
            pltpu.VMEM((2, D, L), jnp.float32),
            pltpu.VMEM((L // _CB, _CB), jnp.float32),
            pltpu.VMEM((SELP, L_K), jnp.float32),
            pltpu.VMEM((L, SELP), jnp.float32),
            pltpu.VMEM((L, _KSP), jnp.float32),
            pltpu.SMEM((n_top,), jnp.int32),
            pltpu.SemaphoreType.DMA((2, 3)),
            pltpu.SemaphoreType.DMA((2,)),
        ],
        compiler_params=pltpu.CompilerParams(
            dimension_semantics=("arbitrary",),
        ),
    )(qT, kT, vT, perm)
    return jnp.transpose(outT, (0, 3, 1, 2))


# speedup vs baseline: 5.8352x; 2.3827x over previous
import functools
import math

import jax
import jax.numpy as jnp
from jax import lax
from jax.experimental import pallas as pl
from jax.experimental.pallas import tpu as pltpu

_FACTOR = 0.0005
_CB = 512
_KSP = 128


def _score_body(U, B, L, H, D,
                q_hbm, k_hbm, perm_ref, m_out,
                qv, kv, ohp_ref, insem):
    NC = L // _CB
    N = B * H
    f32 = jnp.float32
    dot = functools.partial(lax.dot_general, preferred_element_type=f32)

    i = pl.program_id(0)
    buf = lax.rem(i, 2)
    nbuf = 1 - buf

    def in_copies(step, slot):
        bb = step // H
        hh = lax.rem(step, H)
        return (
            pltpu.make_async_copy(q_hbm.at[bb, hh], qv.at[slot],
                                  insem.at[slot, 0]),
            pltpu.make_async_copy(k_hbm.at[bb, hh], kv.at[slot],
                                  insem.at[slot, 1]),
        )

    @pl.when(i == 0)
    def _():
        for c in in_copies(i, buf):
            c.start()
        pv = perm_ref[0, :]
        for c in range(NC):
            lrow = (lax.broadcasted_iota(jnp.int32, (_CB, _KSP), 0)
                    + c * _CB)
            srow = lax.broadcasted_iota(jnp.int32, (_KSP, _KSP), 0)
            pmat = jnp.broadcast_to(pv[None, :], (_CB, _KSP))
            ohp_ref[pl.ds(c * _CB, _CB), :] = jnp.where(
                lrow == pmat, 1.0, 0.0).astype(f32)

    @pl.when(i + 1 < N)
    def _():
        for c in in_copies(i + 1, nbuf):
            c.start()

    cq, ck = in_copies(i, buf)
    cq.wait()
    ck.wait()
    ksT = dot(kv[buf], ohp_ref[...], (((1,), (0,)), ((), ())))
    row = lax.broadcasted_iota(jnp.int32, (_KSP, _CB), 0)
    valid = row < U
    for c in range(NC):
        qc = qv[buf, :, pl.ds(c * _CB, _CB)]
        sc = dot(ksT, qc, (((0,), (0,)), ((), ())))
        mx = jnp.max(jnp.where(valid, sc, -jnp.inf), axis=0)
        sm = jnp.sum(jnp.where(valid, sc, 0.0), axis=0)
        m_out[0, 0, pl.ds(c * _CB, _CB)] = mx - sm / L


def _attn_body(n_top, B, L, H, D, scale,
               m_hbm, q_hbm, k_hbm, v_hbm, out_hbm,
               qv, kv, vv, ov, mw_ref, att_ref, oh_ref, idx_ref,
               insem, outsem, msem):
    L_K = L
    NC = L // _CB
    N = B * H
    f32 = jnp.float32
    SELP = idx_ref.shape[1]
    dot = functools.partial(lax.dot_general, preferred_element_type=f32)

    i = pl.program_id(0)
    buf = lax.rem(i, 2)
    nbuf = 1 - buf
    b = i // H
    h = lax.rem(i, H)

    def in_copies(step, slot):
        bb = step // H
        hh = lax.rem(step, H)
        return (
            pltpu.make_async_copy(q_hbm.at[bb, hh], qv.at[slot],
                                  insem.at[slot, 0]),
            pltpu.make_async_copy(k_hbm.at[bb, hh], kv.at[slot],
                                  insem.at[slot, 1]),
            pltpu.make_async_copy(v_hbm.at[bb, hh], vv.at[slot],
                                  insem.at[slot, 2]),
        )

    @pl.when(i == 0)
    def _():
        cm = pltpu.make_async_copy(m_hbm.at[:, 0, :], mw_ref, msem)
        cm.start()
        for c in in_copies(i, buf):
            c.start()
        cm.wait()
        lanes = lax.broadcasted_iota(jnp.int32, (N, L), 1)
        for t in range(n_top):
            mc = mw_ref[...]
            mrow = jnp.max(mc, axis=1, keepdims=True)
            cand = jnp.where(mc == mrow, lanes, jnp.int32(2 ** 30))
            idxr = jnp.min(cand, axis=1, keepdims=True)
            idx_ref[:, t:t + 1] = idxr
            mw_ref[...] = jnp.where(lanes == idxr, -jnp.inf, mc)
        if SELP > n_top:
            idx_ref[:, n_top:] = jnp.full((N, SELP - n_top), 2 ** 30,
                                          jnp.int32)

    @pl.when(i + 1 < N)
    def _():
        for c in in_copies(i + 1, nbuf):
            c.start()

    cq, ck, cv = in_copies(i, buf)

    idxv = idx_ref[pl.ds(i, 1), :]
    cq.wait()
    for c in range(NC):
        lsub = (lax.broadcasted_iota(jnp.int32, (_CB, SELP), 0) + c * _CB)
        im = jnp.broadcast_to(idxv, (_CB, SELP))
        oh_ref[pl.ds(c * _CB, _CB), :] = jnp.where(
            lsub == im, 1.0, 0.0).astype(f32)
    oh = oh_ref[...]
    selq = dot(qv[buf], oh, (((1,), (0,)), ((), ())))

    ck.wait()
    for c in range(NC):
        kc = kv[buf, :, pl.ds(c * _CB, _CB)]
        att_ref[:, pl.ds(c * _CB, _CB)] = (
            dot(selq, kc, (((0,), (0,)), ((), ()))) * scale)
    for r in range(SELP // 8):
        a = att_ref[pl.ds(r * 8, 8), :]
        a = a - jnp.max(a, axis=1, keepdims=True)
        e = jnp.exp(a)
        att_ref[pl.ds(r * 8, 8), :] = e / jnp.sum(e, axis=1, keepdims=True)
    cv.wait()
    ctx = dot(vv[buf], att_ref[...], (((1,), (1,)), ((), ())))

    def out_copy(slot, bb, hh):
        return pltpu.make_async_copy(ov.at[slot], out_hbm.at[bb, hh],
                                     outsem.at[slot])

    @pl.when(i >= 2)
    def _():
        out_copy(buf, b, h).wait()

    for c in range(NC):
        ohc = oh_ref[pl.ds(c * _CB, _CB), :]
        ov[buf, :, pl.ds(c * _CB, _CB)] = (
            dot(ctx, ohc, (((1,), (1,)), ((), ()))))
    out_copy(buf, b, h).start()

    @pl.when(i == N - 1)
    def _():
        out_copy(buf, b, h).wait()
        @pl.when(N >= 2)
        def _():
            out_copy(nbuf, b, h).wait()


def kernel(queries, keys, values):
    B, L, H, D = queries.shape
    L_K = keys.shape[1]
    scale = 1.0 / math.sqrt(D)
    n_top = max(int(L * _FACTOR * math.log(L_K)), 1)
    U = min(n_top, L_K)
    SELP = (n_top + 7) // 8 * 8
    N = B * H

    qT = jnp.transpose(queries, (0, 2, 3, 1))
    kT = jnp.transpose(keys, (0, 2, 3, 1))
    vT = jnp.transpose(values, (0, 2, 3, 1))

    perm = jax.random.permutation(jax.random.key(42), L_K)[:U]
    perm = jnp.pad(perm.astype(jnp.int32), (0, _KSP - U),
                   constant_values=-1)[None, :]

    any_spec = pl.BlockSpec(memory_space=pl.ANY)
    m_all = pl.pallas_call(
        functools.partial(_score_body, U, B, L, H, D),
        grid=(N,),
        in_specs=[
            any_spec,
            any_spec,
            pl.BlockSpec((1, _KSP), lambda i: (0, 0)),
        ],
        out_specs=pl.BlockSpec((1, 1, L), lambda i: (i, 0, 0)),
        out_shape=jax.ShapeDtypeStruct((N, 1, L), jnp.float32),
        scratch_shapes=[
            pltpu.VMEM((2, D, L), jnp.float32),
            pltpu.VMEM((2, D, L), jnp.float32),
            pltpu.VMEM((L, _KSP), jnp.float32),
            pltpu.SemaphoreType.DMA((2, 2)),
        ],
        compiler_params=pltpu.CompilerParams(
            dimension_semantics=("arbitrary",),
        ),
    )(qT, kT, perm)

    outT = pl.pallas_call(
        functools.partial(_attn_body, n_top, B, L, H, D, scale),
        grid=(N,),
        in_specs=[
            any_spec,
            any_spec,
            any_spec,
            any_spec,
        ],
        out_specs=any_spec,
        out_shape=jax.ShapeDtypeStruct((B, H, D, L), jnp.float32),
        scratch_shapes=[
            pltpu.VMEM((2, D, L), jnp.float32),
            pltpu.VMEM((2, D, L), jnp.float32),
            pltpu.VMEM((2, D, L), jnp.float32),
            pltpu.VMEM((2, D, L), jnp.float32),
            pltpu.VMEM((N, L), jnp.float32),
            pltpu.VMEM((SELP, L), jnp.float32),
            pltpu.VMEM((L, SELP), jnp.float32),
            pltpu.VMEM((N, SELP), jnp.int32),
            pltpu.SemaphoreType.DMA((2, 3)),
            pltpu.SemaphoreType.DMA((2,)),
            pltpu.SemaphoreType.DMA,
        ],
        compiler_params=pltpu.CompilerParams(
            dimension_semantics=("arbitrary",),
        ),
    )(m_all, qT, kT, vT)
    return jnp.transpose(outT, (0, 3, 1, 2))
